# Initial kernel scaffold; baseline (speedup 1.0000x reference)
import jax
import jax.numpy as jnp
from jax.experimental import pallas as pl


def kernel(x, edge_index, pe, Wl1, bl1, Wr1, Wl2, bl2, Wr2, W_fc1, b_fc1, g1, be1, W_fc2, b_fc2, g2, be2, W_fc3, b_fc3):
    raise NotImplementedError("write your pallas kernel here")


# speedup vs baseline: 5.7373x; 5.7373x over previous
import functools
import math

import jax
import jax.numpy as jnp
from jax import lax
from jax.experimental import pallas as pl
from jax.experimental.pallas import tpu as pltpu
from jax.experimental.pallas import tpu_sc as plsc

N_GRAPHS = 32
NODES_PER_GRAPH = 1008
N = N_GRAPHS * NODES_PER_GRAPH
E = N * 16
NC, NS = 2, 16
EPAD = 524288
CH = 128
NCH = EPAD // NS // CH
NPAD = 512
CPB = 32
NIB = NCH // CPB
ZROWS = (N + NPAD) // NS
WROWS = N // NS


def _sc_segsum_body(table, src_idx, dst_idx, zer, out, src_v, dst_v, rows_v, acc, sem):
    c = lax.axis_index("c")
    s = lax.axis_index("s")
    pltpu.sync_copy(zer, acc.at[pl.ds(s * ZROWS, ZROWS)])
    plsc.subcore_barrier()

    def iblock(b, carry):
        pltpu.sync_copy(src_idx.at[c, s, pl.ds(b * CPB, CPB)], src_v)
        pltpu.sync_copy(dst_idx.at[s, pl.ds(b * CPB, CPB)], dst_v)

        def chunk(i, carry2):
            pltpu.async_copy(table.at[src_v.at[i]], rows_v, sem).wait()
            pltpu.sync_copy(rows_v, acc.at[dst_v.at[i]], add=True)
            return carry2

        return lax.fori_loop(0, CPB, chunk, carry)

    lax.fori_loop(0, NIB, iblock, 0)
    plsc.subcore_barrier()
    pltpu.sync_copy(acc.at[pl.ds(s * WROWS, WROWS)], out.at[c, pl.ds(s * WROWS, WROWS)])


def _sc_segsum_l1(table, src_idx, dst_idx, zer):
    W = 48
    return pl.kernel(
        _sc_segsum_body,
        out_type=jax.ShapeDtypeStruct((NC, N, W), jnp.float32),
        mesh=plsc.VectorSubcoreMesh(core_axis_name="c", subcore_axis_name="s"),
        scratch_types=[
            pltpu.VMEM((CPB, CH), jnp.int32),
            pltpu.VMEM((CPB, CH), jnp.int32),
            pltpu.VMEM((CH, W), jnp.float32),
            pltpu.VMEM_SHARED((N + NPAD, W), jnp.float32),
            pltpu.SemaphoreType.DMA,
        ],
        compiler_params=pltpu.CompilerParams(use_tc_tiling_on_sc=False),
    )(table, src_idx, dst_idx, zer)


def _sc_segsum2_body(table, src_idx, dst_idx, zer, out, src_v, dst_v, rows_v, acc, sem):
    c = lax.axis_index("c")
    s = lax.axis_index("s")
    for p in range(2):
        q = 2 * c + p
        pltpu.sync_copy(zer, acc.at[pl.ds(s * ZROWS, ZROWS)])
        plsc.subcore_barrier()

        def iblock(b, carry):
            pltpu.sync_copy(src_idx.at[q, s, pl.ds(b * CPB, CPB)], src_v)
            pltpu.sync_copy(dst_idx.at[s, pl.ds(b * CPB, CPB)], dst_v)

            def chunk(i, carry2):
                pltpu.async_copy(table.at[src_v.at[i]], rows_v, sem).wait()
                pltpu.sync_copy(rows_v, acc.at[dst_v.at[i]], add=True)
                return carry2

            return lax.fori_loop(0, CPB, chunk, carry)

        lax.fori_loop(0, NIB, iblock, 0)
        plsc.subcore_barrier()
        pltpu.sync_copy(acc.at[pl.ds(s * WROWS, WROWS)], out.at[q, pl.ds(s * WROWS, WROWS)])
        plsc.subcore_barrier()


def _sc_segsum_l2(table, src_idx, dst_idx, zer):
    W = 32
    return pl.kernel(
        _sc_segsum2_body,
        out_type=jax.ShapeDtypeStruct((4, N, W), jnp.float32),
        mesh=plsc.VectorSubcoreMesh(core_axis_name="c", subcore_axis_name="s"),
        scratch_types=[
            pltpu.VMEM((CPB, CH), jnp.int32),
            pltpu.VMEM((CPB, CH), jnp.int32),
            pltpu.VMEM((CH, W), jnp.float32),
            pltpu.VMEM_SHARED((N + NPAD, W), jnp.float32),
            pltpu.SemaphoreType.DMA,
        ],
        compiler_params=pltpu.CompilerParams(use_tc_tiling_on_sc=False),
    )(table, src_idx, dst_idx, zer)


def _tc_h1_body(o0, o1, x, wl, wr, bl, h1_out, h1q_out):
    deg = o0[:, 42:43]
    rd = 1.0 / jnp.maximum(deg, 1.0)
    a0 = o0[:, :42] * rd
    a1 = o1[:, :42] * rd
    h1 = (
        jnp.dot(a0, wl[:42], preferred_element_type=jnp.float32)
        + jnp.dot(a1, wl[42:84], preferred_element_type=jnp.float32)
        + jnp.dot(x[...], wr[...], preferred_element_type=jnp.float32)
        + bl[...]
    )
    h1_out[...] = h1
    h1q_out[...] = jnp.transpose(h1.reshape(h1.shape[0], 4, 32), (1, 0, 2))


def _tc_h1(o0, o1, x, wl, wr, bl):
    blk = NODES_PER_GRAPH
    grid = (N // blk,)
    return pl.pallas_call(
        _tc_h1_body,
        grid=grid,
        in_specs=[
            pl.BlockSpec((blk, 48), lambda i: (i, 0)),
            pl.BlockSpec((blk, 48), lambda i: (i, 0)),
            pl.BlockSpec((blk, 84), lambda i: (i, 0)),
            pl.BlockSpec((84, 128), lambda i: (0, 0)),
            pl.BlockSpec((84, 128), lambda i: (0, 0)),
            pl.BlockSpec((1, 128), lambda i: (0, 0)),
        ],
        out_specs=[
            pl.BlockSpec((blk, 128), lambda i: (i, 0)),
            pl.BlockSpec((4, blk, 32), lambda i: (0, i, 0)),
        ],
        out_shape=[
            jax.ShapeDtypeStruct((N, 128), jnp.float32),
            jax.ShapeDtypeStruct((4, N, 32), jnp.float32),
        ],
    )(o0, o1, x, wl, wr, bl)


def _tc_att_body(o2, o0, h1, pe, wl, wr, bl, aggr_out):
    agg2 = jnp.concatenate([o2[0], o2[1], o2[2], o2[3]], axis=-1)
    deg = o0[:, 42:43]
    rd = 1.0 / jnp.maximum(deg, 1.0)
    h1v = h1[...]
    h2 = (
        jnp.dot(agg2 * rd, wl[...], preferred_element_type=jnp.float32)
        + jnp.dot(h1v, wr[...], preferred_element_type=jnp.float32)
        + bl[...]
    )
    t2 = h2 + pe[...]
    score = lax.dot_general(t2, t2, (((1,), (1,)), ((), ())),
                            preferred_element_type=jnp.float32) * (1.0 / math.sqrt(128.0))
    m = jnp.max(score, axis=-1, keepdims=True)
    p = jnp.exp(score - m)
    attn = p / jnp.sum(p, axis=-1, keepdims=True)
    ctx = jnp.dot(attn, t2, preferred_element_type=jnp.float32)

    parts = []
    for tag in range(2):
        srcm = h1v if tag == 0 else ctx
        maxs = []
        means = []
        for j in range(12):
            blkv = srcm[84 * j:84 * (j + 1), :]
            maxs.append(jnp.max(blkv, axis=0, keepdims=True))
            means.append(jnp.mean(blkv, axis=0, keepdims=True))
        parts.append((jnp.concatenate(maxs, axis=0), jnp.concatenate(means, axis=0)))
    combined = jnp.concatenate([parts[0][0], parts[0][1], parts[1][0], parts[1][1]], axis=-1)
    aggr_out[...] = combined.reshape(1, 1, 6144)


def _tc_att(o2, o0, h1, pe, wl, wr, bl):
    blk = NODES_PER_GRAPH
    return pl.pallas_call(
        _tc_att_body,
        grid=(N_GRAPHS,),
        in_specs=[
            pl.BlockSpec((4, blk, 32), lambda g: (0, g, 0)),
            pl.BlockSpec((blk, 48), lambda g: (g, 0)),
            pl.BlockSpec((blk, 128), lambda g: (g, 0)),
            pl.BlockSpec((blk, 128), lambda g: (0, 0)),
            pl.BlockSpec((128, 128), lambda g: (0, 0)),
            pl.BlockSpec((128, 128), lambda g: (0, 0)),
            pl.BlockSpec((1, 128), lambda g: (0, 0)),
        ],
        out_specs=pl.BlockSpec((1, 1, 6144), lambda g: (g, 0, 0)),
        out_shape=jax.ShapeDtypeStruct((N_GRAPHS, 1, 6144), jnp.float32),
    )(o2, o0, h1, pe, wl, wr, bl)


def _tc_mlp_body(aggr, w1, b1, g1, be1, w2, b2, g2, be2, w3, b3, out):
    def bn(z, g, b):
        mm = jnp.mean(z, axis=0, keepdims=True)
        v = jnp.mean((z - mm) ** 2, axis=0, keepdims=True)
        return (z - mm) / jnp.sqrt(v + 1e-5) * g + b

    def silu(z):
        return z / (1.0 + jnp.exp(-z))

    z = silu(jnp.dot(aggr[...], w1[...], preferred_element_type=jnp.float32) + b1[...])
    z = bn(z, g1[...], be1[...])
    z = silu(jnp.dot(z, w2[...], preferred_element_type=jnp.float32) + b2[...])
    z = bn(z, g2[...], be2[...])
    logits = jnp.dot(z, w3[...], preferred_element_type=jnp.float32) + b3[...]
    l2 = logits[:, :2]
    lm = jnp.max(l2, axis=1, keepdims=True)
    e = jnp.exp(l2 - lm)
    sm = e / jnp.sum(e, axis=1, keepdims=True)
    out[...] = jnp.concatenate([sm, jnp.zeros((sm.shape[0], 126), jnp.float32)], axis=1)


def _tc_mlp(aggr, w1, b1, g1, be1, w2, b2, g2, be2, w3, b3):
    return pl.pallas_call(
        _tc_mlp_body,
        out_shape=jax.ShapeDtypeStruct((N_GRAPHS, 128), jnp.float32),
    )(aggr, w1, b1, g1, be1, w2, b2, g2, be2, w3, b3)


def kernel(x, edge_index, pe, Wl1, bl1, Wr1, Wl2, bl2, Wr2, W_fc1, b_fc1, g1,
           be1, W_fc2, b_fc2, g2, be2, W_fc3, b_fc3):
    f32 = jnp.float32
    src = edge_index[0]
    dst = edge_index[1]
    npad = EPAD - E
    src_pad = (jnp.arange(npad, dtype=jnp.int32) * 63) % N
    dst_pad = N + (jnp.arange(npad, dtype=jnp.int32) % NPAD)
    srcr = jnp.concatenate([src, src_pad]).reshape(NS, NCH, CH)
    dstr = jnp.concatenate([dst, dst_pad]).reshape(NS, NCH, CH)
    src_l1 = jnp.stack([srcr, srcr + N])
    src_l2 = jnp.stack([srcr + q * N for q in range(4)])

    ones = jnp.ones((N, 1), f32)
    t0 = jnp.concatenate([x[:, :42], ones, jnp.zeros((N, 5), f32)], axis=1)
    t1 = jnp.concatenate([x[:, 42:84], jnp.zeros((N, 6), f32)], axis=1)
    table1 = jnp.concatenate([t0, t1], axis=0)
    zer48 = jnp.zeros((ZROWS, 48), f32)
    zer32 = jnp.zeros((ZROWS, 32), f32)

    oA = _sc_segsum_l1(table1, src_l1, dstr, zer48)

    h1, h1q = _tc_h1(oA[0], oA[1], x, Wl1, Wr1, bl1.reshape(1, 128))

    o2 = _sc_segsum_l2(h1q.reshape(4 * N, 32), src_l2, dstr, zer32)

    aggr = _tc_att(o2, oA[0], h1, pe, Wl2, Wr2, bl2.reshape(1, 128)).reshape(N_GRAPHS, 6144)

    w2p = jnp.pad(W_fc2, ((0, 0), (0, 96)))
    b2p = jnp.pad(b_fc2, (0, 96)).reshape(1, 128)
    g2p = jnp.pad(g2, (0, 96)).reshape(1, 128)
    be2p = jnp.pad(be2, (0, 96)).reshape(1, 128)
    w3p = jnp.pad(W_fc3, ((0, 96), (0, 126)))
    b3p = jnp.pad(b_fc3, (0, 126)).reshape(1, 128)
    outp = _tc_mlp(aggr, W_fc1, b_fc1.reshape(1, 512), g1.reshape(1, 512),
                   be1.reshape(1, 512), w2p, b2p, g2p, be2p, w3p, b3p)
    return outp[:, :2]


# speedup vs baseline: 8.1474x; 1.4201x over previous
import functools
import math

import jax
import jax.numpy as jnp
from jax import lax
from jax.experimental import pallas as pl
from jax.experimental.pallas import tpu as pltpu
from jax.experimental.pallas import tpu_sc as plsc

N_GRAPHS = 32
NODES_PER_GRAPH = 1008
N = N_GRAPHS * NODES_PER_GRAPH
E = N * 16
NC, NS = 2, 16
EPAD = 524288
CH = 128
NCH = EPAD // NS // CH
NPAD = 512
CPB = 32
NIB = NCH // CPB
ZROWS = (N + NPAD) // NS
WROWS = N // NS


def _sc_edge_blocks(table, src_idx_plane, dst_idx, acc, src_v, dst_v, rows0, rows1, sem0, sem1):
    s = lax.axis_index("s")

    def iblock(b, carry):
        pltpu.sync_copy(src_idx_plane.at[s, pl.ds(b * CPB, CPB)], src_v)
        pltpu.sync_copy(dst_idx.at[s, pl.ds(b * CPB, CPB)], dst_v)
        pltpu.async_copy(table.at[src_v.at[0]], rows0, sem0)

        def pair(k, carry2):
            i0 = 2 * k
            pltpu.async_copy(table.at[src_v.at[i0 + 1]], rows1, sem1)
            pltpu.make_async_copy(table.at[src_v.at[i0]], rows0, sem0).wait()
            pltpu.sync_copy(rows0, acc.at[dst_v.at[i0]], add=True)
            pltpu.async_copy(table.at[src_v.at[i0 + 2]], rows0, sem0)
            pltpu.make_async_copy(table.at[src_v.at[i0 + 1]], rows1, sem1).wait()
            pltpu.sync_copy(rows1, acc.at[dst_v.at[i0 + 1]], add=True)
            return carry2

        lax.fori_loop(0, CPB // 2 - 1, pair, carry)
        pltpu.async_copy(table.at[src_v.at[CPB - 1]], rows1, sem1)
        pltpu.make_async_copy(table.at[src_v.at[CPB - 2]], rows0, sem0).wait()
        pltpu.sync_copy(rows0, acc.at[dst_v.at[CPB - 2]], add=True)
        pltpu.make_async_copy(table.at[src_v.at[CPB - 1]], rows1, sem1).wait()
        pltpu.sync_copy(rows1, acc.at[dst_v.at[CPB - 1]], add=True)
        return carry

    lax.fori_loop(0, NIB, iblock, 0)


def _sc_segsum_body(table, src_idx, dst_idx, zer, out, src_v, dst_v, rows0, rows1, acc, sem0, sem1):
    c = lax.axis_index("c")
    s = lax.axis_index("s")
    pltpu.sync_copy(zer, acc.at[pl.ds(s * ZROWS, ZROWS)])
    plsc.subcore_barrier()
    _sc_edge_blocks(table, src_idx.at[c], dst_idx, acc, src_v, dst_v, rows0, rows1, sem0, sem1)
    plsc.subcore_barrier()
    pltpu.sync_copy(acc.at[pl.ds(s * WROWS, WROWS)], out.at[c, pl.ds(s * WROWS, WROWS)])


def _sc_segsum_l1(table, src_idx, dst_idx, zer):
    W = 48
    return pl.kernel(
        _sc_segsum_body,
        out_type=jax.ShapeDtypeStruct((NC, N, W), jnp.float32),
        mesh=plsc.VectorSubcoreMesh(core_axis_name="c", subcore_axis_name="s"),
        scratch_types=[
            pltpu.VMEM((CPB, CH), jnp.int32),
            pltpu.VMEM((CPB, CH), jnp.int32),
            pltpu.VMEM((CH, W), jnp.float32),
            pltpu.VMEM((CH, W), jnp.float32),
            pltpu.VMEM_SHARED((N + NPAD, W), jnp.float32),
            pltpu.SemaphoreType.DMA,
            pltpu.SemaphoreType.DMA,
        ],
        compiler_params=pltpu.CompilerParams(use_tc_tiling_on_sc=False),
    )(table, src_idx, dst_idx, zer)


def _sc_segsum2_body(table, src_idx, dst_idx, zer, out, src_v, dst_v, rows0, rows1, acc, sem0, sem1):
    c = lax.axis_index("c")
    s = lax.axis_index("s")
    for p in range(2):
        q = 2 * c + p
        pltpu.sync_copy(zer, acc.at[pl.ds(s * ZROWS, ZROWS)])
        plsc.subcore_barrier()
        _sc_edge_blocks(table, src_idx.at[q], dst_idx, acc, src_v, dst_v, rows0, rows1, sem0, sem1)
        plsc.subcore_barrier()
        pltpu.sync_copy(acc.at[pl.ds(s * WROWS, WROWS)], out.at[q, pl.ds(s * WROWS, WROWS)])
        plsc.subcore_barrier()


def _sc_segsum_l2(table, src_idx, dst_idx, zer):
    W = 32
    return pl.kernel(
        _sc_segsum2_body,
        out_type=jax.ShapeDtypeStruct((4, N, W), jnp.float32),
        mesh=plsc.VectorSubcoreMesh(core_axis_name="c", subcore_axis_name="s"),
        scratch_types=[
            pltpu.VMEM((CPB, CH), jnp.int32),
            pltpu.VMEM((CPB, CH), jnp.int32),
            pltpu.VMEM((CH, W), jnp.float32),
            pltpu.VMEM((CH, W), jnp.float32),
            pltpu.VMEM_SHARED((N + NPAD, W), jnp.float32),
            pltpu.SemaphoreType.DMA,
            pltpu.SemaphoreType.DMA,
        ],
        compiler_params=pltpu.CompilerParams(use_tc_tiling_on_sc=False),
    )(table, src_idx, dst_idx, zer)


def _tc_h1_body(o0, o1, x, wl, wr, bl, h1_out, h1q_out):
    deg = o0[:, 42:43]
    rd = 1.0 / jnp.maximum(deg, 1.0)
    a0 = o0[:, :42] * rd
    a1 = o1[:, :42] * rd
    h1 = (
        jnp.dot(a0, wl[:42], preferred_element_type=jnp.float32)
        + jnp.dot(a1, wl[42:84], preferred_element_type=jnp.float32)
        + jnp.dot(x[...], wr[...], preferred_element_type=jnp.float32)
        + bl[...]
    )
    h1_out[...] = h1
    h1q_out[...] = jnp.transpose(h1.reshape(h1.shape[0], 4, 32), (1, 0, 2))


def _tc_h1(o0, o1, x, wl, wr, bl):
    blk = NODES_PER_GRAPH
    grid = (N // blk,)
    return pl.pallas_call(
        _tc_h1_body,
        grid=grid,
        in_specs=[
            pl.BlockSpec((blk, 48), lambda i: (i, 0)),
            pl.BlockSpec((blk, 48), lambda i: (i, 0)),
            pl.BlockSpec((blk, 84), lambda i: (i, 0)),
            pl.BlockSpec((84, 128), lambda i: (0, 0)),
            pl.BlockSpec((84, 128), lambda i: (0, 0)),
            pl.BlockSpec((1, 128), lambda i: (0, 0)),
        ],
        out_specs=[
            pl.BlockSpec((blk, 128), lambda i: (i, 0)),
            pl.BlockSpec((4, blk, 32), lambda i: (0, i, 0)),
        ],
        out_shape=[
            jax.ShapeDtypeStruct((N, 128), jnp.float32),
            jax.ShapeDtypeStruct((4, N, 32), jnp.float32),
        ],
    )(o0, o1, x, wl, wr, bl)


def _tc_att_body(o2, o0, h1, pe, wl, wr, bl, aggr_out):
    agg2 = jnp.concatenate([o2[0], o2[1], o2[2], o2[3]], axis=-1)
    deg = o0[:, 42:43]
    rd = 1.0 / jnp.maximum(deg, 1.0)
    h1v = h1[...]
    h2 = (
        jnp.dot(agg2 * rd, wl[...], preferred_element_type=jnp.float32)
        + jnp.dot(h1v, wr[...], preferred_element_type=jnp.float32)
        + bl[...]
    )
    t2 = h2 + pe[...]
    score = lax.dot_general(t2, t2, (((1,), (1,)), ((), ())),
                            preferred_element_type=jnp.float32) * (1.0 / math.sqrt(128.0))
    m = jnp.max(score, axis=-1, keepdims=True)
    p = jnp.exp(score - m)
    attn = p / jnp.sum(p, axis=-1, keepdims=True)
    ctx = jnp.dot(attn, t2, preferred_element_type=jnp.float32)

    parts = []
    for tag in range(2):
        srcm = h1v if tag == 0 else ctx
        maxs = []
        means = []
        for j in range(12):
            blkv = srcm[84 * j:84 * (j + 1), :]
            maxs.append(jnp.max(blkv, axis=0, keepdims=True))
            means.append(jnp.mean(blkv, axis=0, keepdims=True))
        parts.append((jnp.concatenate(maxs, axis=0), jnp.concatenate(means, axis=0)))
    combined = jnp.concatenate([parts[0][0], parts[0][1], parts[1][0], parts[1][1]], axis=-1)
    aggr_out[...] = combined.reshape(1, 1, 6144)


def _tc_att(o2, o0, h1, pe, wl, wr, bl):
    blk = NODES_PER_GRAPH
    return pl.pallas_call(
        _tc_att_body,
        grid=(N_GRAPHS,),
        in_specs=[
            pl.BlockSpec((4, blk, 32), lambda g: (0, g, 0)),
            pl.BlockSpec((blk, 48), lambda g: (g, 0)),
            pl.BlockSpec((blk, 128), lambda g: (g, 0)),
            pl.BlockSpec((blk, 128), lambda g: (0, 0)),
            pl.BlockSpec((128, 128), lambda g: (0, 0)),
            pl.BlockSpec((128, 128), lambda g: (0, 0)),
            pl.BlockSpec((1, 128), lambda g: (0, 0)),
        ],
        out_specs=pl.BlockSpec((1, 1, 6144), lambda g: (g, 0, 0)),
        out_shape=jax.ShapeDtypeStruct((N_GRAPHS, 1, 6144), jnp.float32),
    )(o2, o0, h1, pe, wl, wr, bl)


def _tc_mlp_body(aggr, w1, b1, g1, be1, w2, b2, g2, be2, w3, b3, out):
    def bn(z, g, b):
        mm = jnp.mean(z, axis=0, keepdims=True)
        v = jnp.mean((z - mm) ** 2, axis=0, keepdims=True)
        return (z - mm) / jnp.sqrt(v + 1e-5) * g + b

    def silu(z):
        return z / (1.0 + jnp.exp(-z))

    z = silu(jnp.dot(aggr[...], w1[...], preferred_element_type=jnp.float32) + b1[...])
    z = bn(z, g1[...], be1[...])
    z = silu(jnp.dot(z, w2[...], preferred_element_type=jnp.float32) + b2[...])
    z = bn(z, g2[...], be2[...])
    logits = jnp.dot(z, w3[...], preferred_element_type=jnp.float32) + b3[...]
    l2 = logits[:, :2]
    lm = jnp.max(l2, axis=1, keepdims=True)
    e = jnp.exp(l2 - lm)
    sm = e / jnp.sum(e, axis=1, keepdims=True)
    out[...] = jnp.concatenate([sm, jnp.zeros((sm.shape[0], 126), jnp.float32)], axis=1)


def _tc_mlp(aggr, w1, b1, g1, be1, w2, b2, g2, be2, w3, b3):
    return pl.pallas_call(
        _tc_mlp_body,
        out_shape=jax.ShapeDtypeStruct((N_GRAPHS, 128), jnp.float32),
    )(aggr, w1, b1, g1, be1, w2, b2, g2, be2, w3, b3)


def kernel(x, edge_index, pe, Wl1, bl1, Wr1, Wl2, bl2, Wr2, W_fc1, b_fc1, g1,
           be1, W_fc2, b_fc2, g2, be2, W_fc3, b_fc3):
    f32 = jnp.float32
    src = edge_index[0]
    dst = edge_index[1]
    npad = EPAD - E
    src_pad = (jnp.arange(npad, dtype=jnp.int32) * 63) % N
    dst_pad = N + (jnp.arange(npad, dtype=jnp.int32) % NPAD)
    srcr = jnp.concatenate([src, src_pad]).reshape(NS, NCH, CH)
    dstr = jnp.concatenate([dst, dst_pad]).reshape(NS, NCH, CH)
    src_l1 = jnp.stack([srcr, srcr + N])
    src_l2 = jnp.stack([srcr + q * N for q in range(4)])

    ones = jnp.ones((N, 1), f32)
    t0 = jnp.concatenate([x[:, :42], ones, jnp.zeros((N, 5), f32)], axis=1)
    t1 = jnp.concatenate([x[:, 42:84], jnp.zeros((N, 6), f32)], axis=1)
    table1 = jnp.concatenate([t0, t1], axis=0)
    zer48 = jnp.zeros((ZROWS, 48), f32)
    zer32 = jnp.zeros((ZROWS, 32), f32)

    oA = _sc_segsum_l1(table1, src_l1, dstr, zer48)

    h1, h1q = _tc_h1(oA[0], oA[1], x, Wl1, Wr1, bl1.reshape(1, 128))

    o2 = _sc_segsum_l2(h1q.reshape(4 * N, 32), src_l2, dstr, zer32)

    aggr = _tc_att(o2, oA[0], h1, pe, Wl2, Wr2, bl2.reshape(1, 128)).reshape(N_GRAPHS, 6144)

    w2p = jnp.pad(W_fc2, ((0, 0), (0, 96)))
    b2p = jnp.pad(b_fc2, (0, 96)).reshape(1, 128)
    g2p = jnp.pad(g2, (0, 96)).reshape(1, 128)
    be2p = jnp.pad(be2, (0, 96)).reshape(1, 128)
    w3p = jnp.pad(W_fc3, ((0, 96), (0, 126)))
    b3p = jnp.pad(b_fc3, (0, 126)).reshape(1, 128)
    outp = _tc_mlp(aggr, W_fc1, b_fc1.reshape(1, 512), g1.reshape(1, 512),
                   be1.reshape(1, 512), w2p, b2p, g2p, be2p, w3p, b3p)
    return outp[:, :2]


# speedup vs baseline: 8.7131x; 1.0694x over previous
import functools
import math

import jax
import jax.numpy as jnp
from jax import lax
from jax.experimental import pallas as pl
from jax.experimental.pallas import tpu as pltpu
from jax.experimental.pallas import tpu_sc as plsc

N_GRAPHS = 32
NODES_PER_GRAPH = 1008
N = N_GRAPHS * NODES_PER_GRAPH
E = N * 16
NC, NS = 2, 16
EPAD = 524288
CH = 128
NCH = EPAD // NS // CH
NPAD = 384
CPB = 32
NIB = NCH // CPB
NIB_GROUPS = CPB // 4
ZROWS = (N + NPAD) // NS
WROWS = N // NS


def _sc_edge_blocks(table, src_idx_plane, dst_idx, acc, src_v, dst_v, rows, gsem, ssem):
    s = lax.axis_index("s")

    def gath(i, t):
        pltpu.async_copy(table.at[src_v.at[i]], rows[t], gsem[t])

    def wait_g(t):
        pltpu.make_async_copy(table.at[src_v.at[0]], rows[t], gsem[t]).wait()

    def scat(i, t):
        pltpu.async_copy(rows[t], acc.at[dst_v.at[i]], ssem[t], add=True)

    def wait_s(t):
        pltpu.make_async_copy(rows[t], acc.at[dst_v.at[0]], ssem[t]).wait()

    def iblock(b, carry):
        pltpu.sync_copy(src_idx_plane.at[s, pl.ds(b * CPB, CPB)], src_v)
        pltpu.sync_copy(dst_idx.at[s, pl.ds(b * CPB, CPB)], dst_v)
        gath(0, 0)
        gath(1, 1)
        for t in range(4):
            wait_g(t)
            scat(t, t)
            if t < 2:
                gath(t + 2, t + 2)
            else:
                wait_s(t - 2)
                gath(t + 2, t - 2)

        def group(j, carry2):
            c = 4 * j
            for t in range(4):
                wait_g(t)
                scat(c + t, t)
                wait_s((t + 2) % 4)
                gath(c + t + 2, (t + 2) % 4)
            return carry2

        lax.fori_loop(1, NIB_GROUPS - 1, group, carry)
        c = CPB - 4
        for t in range(4):
            wait_g(t)
            scat(c + t, t)
            if t < 2:
                wait_s(t + 2)
                gath(c + t + 2, t + 2)
        for t in range(4):
            wait_s(t)
        return carry

    lax.fori_loop(0, NIB, iblock, 0)


def _sc_segsum_body(table, src_idx, dst_idx, zer, out, src_v, dst_v,
                    r0, r1, r2, r3, acc, g0, g1, g2, g3, s0, s1, s2, s3):
    c = lax.axis_index("c")
    s = lax.axis_index("s")
    pltpu.sync_copy(zer, acc.at[pl.ds(s * ZROWS, ZROWS)])
    plsc.subcore_barrier()
    _sc_edge_blocks(table, src_idx.at[c], dst_idx, acc, src_v, dst_v,
                    (r0, r1, r2, r3), (g0, g1, g2, g3), (s0, s1, s2, s3))
    plsc.subcore_barrier()
    pltpu.sync_copy(acc.at[pl.ds(s * WROWS, WROWS)], out.at[c, pl.ds(s * WROWS, WROWS)])


def _sc_segsum_l1(table, src_idx, dst_idx, zer):
    W = 48
    return pl.kernel(
        _sc_segsum_body,
        out_type=jax.ShapeDtypeStruct((NC, N, W), jnp.float32),
        mesh=plsc.VectorSubcoreMesh(core_axis_name="c", subcore_axis_name="s"),
        scratch_types=[
            pltpu.VMEM((CPB, CH), jnp.int32),
            pltpu.VMEM((CPB, CH), jnp.int32),
            pltpu.VMEM((CH, W), jnp.float32),
            pltpu.VMEM((CH, W), jnp.float32),
            pltpu.VMEM((CH, W), jnp.float32),
            pltpu.VMEM((CH, W), jnp.float32),
            pltpu.VMEM_SHARED((N + NPAD, W), jnp.float32),
            pltpu.SemaphoreType.DMA,
            pltpu.SemaphoreType.DMA,
            pltpu.SemaphoreType.DMA,
            pltpu.SemaphoreType.DMA,
            pltpu.SemaphoreType.DMA,
            pltpu.SemaphoreType.DMA,
            pltpu.SemaphoreType.DMA,
            pltpu.SemaphoreType.DMA,
        ],
        compiler_params=pltpu.CompilerParams(use_tc_tiling_on_sc=False),
    )(table, src_idx, dst_idx, zer)


def _sc_segsum2_body(table, src_idx, dst_idx, zer, out, src_v, dst_v,
                     r0, r1, r2, r3, acc, g0, g1, g2, g3, s0, s1, s2, s3):
    c = lax.axis_index("c")
    s = lax.axis_index("s")
    for p in range(2):
        q = 2 * c + p
        pltpu.sync_copy(zer, acc.at[pl.ds(s * ZROWS, ZROWS)])
        plsc.subcore_barrier()
        _sc_edge_blocks(table, src_idx.at[q], dst_idx, acc, src_v, dst_v,
                        (r0, r1, r2, r3), (g0, g1, g2, g3), (s0, s1, s2, s3))
        plsc.subcore_barrier()
        pltpu.sync_copy(acc.at[pl.ds(s * WROWS, WROWS)], out.at[q, pl.ds(s * WROWS, WROWS)])
        plsc.subcore_barrier()


def _sc_segsum_l2(table, src_idx, dst_idx, zer):
    W = 32
    return pl.kernel(
        _sc_segsum2_body,
        out_type=jax.ShapeDtypeStruct((4, N, W), jnp.float32),
        mesh=plsc.VectorSubcoreMesh(core_axis_name="c", subcore_axis_name="s"),
        scratch_types=[
            pltpu.VMEM((CPB, CH), jnp.int32),
            pltpu.VMEM((CPB, CH), jnp.int32),
            pltpu.VMEM((CH, W), jnp.float32),
            pltpu.VMEM((CH, W), jnp.float32),
            pltpu.VMEM((CH, W), jnp.float32),
            pltpu.VMEM((CH, W), jnp.float32),
            pltpu.VMEM_SHARED((N + NPAD, W), jnp.float32),
            pltpu.SemaphoreType.DMA,
            pltpu.SemaphoreType.DMA,
            pltpu.SemaphoreType.DMA,
            pltpu.SemaphoreType.DMA,
            pltpu.SemaphoreType.DMA,
            pltpu.SemaphoreType.DMA,
            pltpu.SemaphoreType.DMA,
            pltpu.SemaphoreType.DMA,
        ],
        compiler_params=pltpu.CompilerParams(use_tc_tiling_on_sc=False),
    )(table, src_idx, dst_idx, zer)


def _tc_h1_body(o0, o1, x, wl, wr, bl, h1_out, h1q_out):
    deg = o0[:, 42:43]
    rd = 1.0 / jnp.maximum(deg, 1.0)
    a0 = o0[:, :42] * rd
    a1 = o1[:, :42] * rd
    h1 = (
        jnp.dot(a0, wl[:42], preferred_element_type=jnp.float32)
        + jnp.dot(a1, wl[42:84], preferred_element_type=jnp.float32)
        + jnp.dot(x[...], wr[...], preferred_element_type=jnp.float32)
        + bl[...]
    )
    h1_out[...] = h1
    h1q_out[...] = jnp.transpose(h1.reshape(h1.shape[0], 4, 32), (1, 0, 2))


def _tc_h1(o0, o1, x, wl, wr, bl):
    blk = NODES_PER_GRAPH
    grid = (N // blk,)
    return pl.pallas_call(
        _tc_h1_body,
        grid=grid,
        in_specs=[
            pl.BlockSpec((blk, 48), lambda i: (i, 0)),
            pl.BlockSpec((blk, 48), lambda i: (i, 0)),
            pl.BlockSpec((blk, 84), lambda i: (i, 0)),
            pl.BlockSpec((84, 128), lambda i: (0, 0)),
            pl.BlockSpec((84, 128), lambda i: (0, 0)),
            pl.BlockSpec((1, 128), lambda i: (0, 0)),
        ],
        out_specs=[
            pl.BlockSpec((blk, 128), lambda i: (i, 0)),
            pl.BlockSpec((4, blk, 32), lambda i: (0, i, 0)),
        ],
        out_shape=[
            jax.ShapeDtypeStruct((N, 128), jnp.float32),
            jax.ShapeDtypeStruct((4, N, 32), jnp.float32),
        ],
    )(o0, o1, x, wl, wr, bl)


def _tc_att_body(o2, o0, h1, pe, wl, wr, bl, aggr_out):
    agg2 = jnp.concatenate([o2[0], o2[1], o2[2], o2[3]], axis=-1)
    deg = o0[:, 42:43]
    rd = 1.0 / jnp.maximum(deg, 1.0)
    h1v = h1[...]
    h2 = (
        jnp.dot(agg2 * rd, wl[...], preferred_element_type=jnp.float32)
        + jnp.dot(h1v, wr[...], preferred_element_type=jnp.float32)
        + bl[...]
    )
    t2 = h2 + pe[...]
    score = lax.dot_general(t2, t2, (((1,), (1,)), ((), ())),
                            preferred_element_type=jnp.float32) * (1.0 / math.sqrt(128.0))
    m = jnp.max(score, axis=-1, keepdims=True)
    p = jnp.exp(score - m)
    attn = p / jnp.sum(p, axis=-1, keepdims=True)
    ctx = jnp.dot(attn, t2, preferred_element_type=jnp.float32)

    parts = []
    for tag in range(2):
        srcm = h1v if tag == 0 else ctx
        maxs = []
        means = []
        for j in range(12):
            blkv = srcm[84 * j:84 * (j + 1), :]
            maxs.append(jnp.max(blkv, axis=0, keepdims=True))
            means.append(jnp.mean(blkv, axis=0, keepdims=True))
        parts.append((jnp.concatenate(maxs, axis=0), jnp.concatenate(means, axis=0)))
    combined = jnp.concatenate([parts[0][0], parts[0][1], parts[1][0], parts[1][1]], axis=-1)
    aggr_out[...] = combined.reshape(1, 1, 6144)


def _tc_att(o2, o0, h1, pe, wl, wr, bl):
    blk = NODES_PER_GRAPH
    return pl.pallas_call(
        _tc_att_body,
        grid=(N_GRAPHS,),
        in_specs=[
            pl.BlockSpec((4, blk, 32), lambda g: (0, g, 0)),
            pl.BlockSpec((blk, 48), lambda g: (g, 0)),
            pl.BlockSpec((blk, 128), lambda g: (g, 0)),
            pl.BlockSpec((blk, 128), lambda g: (0, 0)),
            pl.BlockSpec((128, 128), lambda g: (0, 0)),
            pl.BlockSpec((128, 128), lambda g: (0, 0)),
            pl.BlockSpec((1, 128), lambda g: (0, 0)),
        ],
        out_specs=pl.BlockSpec((1, 1, 6144), lambda g: (g, 0, 0)),
        out_shape=jax.ShapeDtypeStruct((N_GRAPHS, 1, 6144), jnp.float32),
    )(o2, o0, h1, pe, wl, wr, bl)


def _tc_mlp_body(aggr, w1, b1, g1, be1, w2, b2, g2, be2, w3, b3, out):
    def bn(z, g, b):
        mm = jnp.mean(z, axis=0, keepdims=True)
        v = jnp.mean((z - mm) ** 2, axis=0, keepdims=True)
        return (z - mm) / jnp.sqrt(v + 1e-5) * g + b

    def silu(z):
        return z / (1.0 + jnp.exp(-z))

    z = silu(jnp.dot(aggr[...], w1[...], preferred_element_type=jnp.float32) + b1[...])
    z = bn(z, g1[...], be1[...])
    z = silu(jnp.dot(z, w2[...], preferred_element_type=jnp.float32) + b2[...])
    z = bn(z, g2[...], be2[...])
    logits = jnp.dot(z, w3[...], preferred_element_type=jnp.float32) + b3[...]
    l2 = logits[:, :2]
    lm = jnp.max(l2, axis=1, keepdims=True)
    e = jnp.exp(l2 - lm)
    sm = e / jnp.sum(e, axis=1, keepdims=True)
    out[...] = jnp.concatenate([sm, jnp.zeros((sm.shape[0], 126), jnp.float32)], axis=1)


def _tc_mlp(aggr, w1, b1, g1, be1, w2, b2, g2, be2, w3, b3):
    return pl.pallas_call(
        _tc_mlp_body,
        out_shape=jax.ShapeDtypeStruct((N_GRAPHS, 128), jnp.float32),
    )(aggr, w1, b1, g1, be1, w2, b2, g2, be2, w3, b3)


def kernel(x, edge_index, pe, Wl1, bl1, Wr1, Wl2, bl2, Wr2, W_fc1, b_fc1, g1,
           be1, W_fc2, b_fc2, g2, be2, W_fc3, b_fc3):
    f32 = jnp.float32
    src = edge_index[0]
    dst = edge_index[1]
    npad = EPAD - E
    src_pad = (jnp.arange(npad, dtype=jnp.int32) * 63) % N
    dst_pad = N + (jnp.arange(npad, dtype=jnp.int32) % NPAD)
    srcr = jnp.concatenate([src, src_pad]).reshape(NS, NCH, CH)
    dstr = jnp.concatenate([dst, dst_pad]).reshape(NS, NCH, CH)
    src_l1 = jnp.stack([srcr, srcr + N])
    src_l2 = jnp.stack([srcr + q * N for q in range(4)])

    ones = jnp.ones((N, 1), f32)
    t0 = jnp.concatenate([x[:, :42], ones, jnp.zeros((N, 5), f32)], axis=1)
    t1 = jnp.concatenate([x[:, 42:84], jnp.zeros((N, 6), f32)], axis=1)
    table1 = jnp.concatenate([t0, t1], axis=0)
    zer48 = jnp.zeros((ZROWS, 48), f32)
    zer32 = jnp.zeros((ZROWS, 32), f32)

    oA = _sc_segsum_l1(table1, src_l1, dstr, zer48)

    h1, h1q = _tc_h1(oA[0], oA[1], x, Wl1, Wr1, bl1.reshape(1, 128))

    o2 = _sc_segsum_l2(h1q.reshape(4 * N, 32), src_l2, dstr, zer32)

    aggr = _tc_att(o2, oA[0], h1, pe, Wl2, Wr2, bl2.reshape(1, 128)).reshape(N_GRAPHS, 6144)

    w2p = jnp.pad(W_fc2, ((0, 0), (0, 96)))
    b2p = jnp.pad(b_fc2, (0, 96)).reshape(1, 128)
    g2p = jnp.pad(g2, (0, 96)).reshape(1, 128)
    be2p = jnp.pad(be2, (0, 96)).reshape(1, 128)
    w3p = jnp.pad(W_fc3, ((0, 96), (0, 126)))
    b3p = jnp.pad(b_fc3, (0, 126)).reshape(1, 128)
    outp = _tc_mlp(aggr, W_fc1, b_fc1.reshape(1, 512), g1.reshape(1, 512),
                   be1.reshape(1, 512), w2p, b2p, g2p, be2p, w3p, b3p)
    return outp[:, :2]


# speedup vs baseline: 8.7671x; 1.0062x over previous
import functools
import math

import jax
import jax.numpy as jnp
from jax import lax
from jax.experimental import pallas as pl
from jax.experimental.pallas import tpu as pltpu
from jax.experimental.pallas import tpu_sc as plsc

N_GRAPHS = 32
NODES_PER_GRAPH = 1008
N = N_GRAPHS * NODES_PER_GRAPH
E = N * 16
NC, NS = 2, 16
EPAD = 524288
CH = 128
NCH = EPAD // NS // CH
NPAD = 384
CPB = 32
NIB = NCH // CPB
NIB_GROUPS = CPB // 4
ZROWS = (N + NPAD) // NS
WROWS = N // NS


def _sc_edge_blocks(table, src_idx_plane, dst_idx, acc, src_v, dst_v, rows, gsem, ssem):
    s = lax.axis_index("s")

    def gath(i, t):
        pltpu.async_copy(table.at[src_v.at[i]], rows[t], gsem[t])

    def wait_g(t):
        pltpu.make_async_copy(table.at[src_v.at[0]], rows[t], gsem[t]).wait()

    def scat(i, t):
        pltpu.async_copy(rows[t], acc.at[dst_v.at[i]], ssem[t], add=True)

    def wait_s(t):
        pltpu.make_async_copy(rows[t], acc.at[dst_v.at[0]], ssem[t]).wait()

    def iblock(b, carry):
        pltpu.sync_copy(src_idx_plane.at[s, pl.ds(b * CPB, CPB)], src_v)
        pltpu.sync_copy(dst_idx.at[s, pl.ds(b * CPB, CPB)], dst_v)
        gath(0, 0)
        gath(1, 1)
        for t in range(4):
            wait_g(t)
            scat(t, t)
            if t < 2:
                gath(t + 2, t + 2)
            else:
                wait_s(t - 2)
                gath(t + 2, t - 2)

        def group(j, carry2):
            c = 4 * j
            for t in range(4):
                wait_g(t)
                scat(c + t, t)
                wait_s((t + 2) % 4)
                gath(c + t + 2, (t + 2) % 4)
            return carry2

        lax.fori_loop(1, NIB_GROUPS - 1, group, carry)
        c = CPB - 4
        for t in range(4):
            wait_g(t)
            scat(c + t, t)
            if t < 2:
                wait_s(t + 2)
                gath(c + t + 2, t + 2)
        for t in range(4):
            wait_s(t)
        return carry

    lax.fori_loop(0, NIB, iblock, 0)


def _sc_segsum_body(table, src_idx, dst_idx, zer, out, src_v, dst_v,
                    r0, r1, r2, r3, acc, g0, g1, g2, g3, s0, s1, s2, s3):
    c = lax.axis_index("c")
    s = lax.axis_index("s")
    pltpu.sync_copy(zer, acc.at[pl.ds(s * ZROWS, ZROWS)])
    plsc.subcore_barrier()
    _sc_edge_blocks(table, src_idx.at[c], dst_idx, acc, src_v, dst_v,
                    (r0, r1, r2, r3), (g0, g1, g2, g3), (s0, s1, s2, s3))
    plsc.subcore_barrier()
    pltpu.sync_copy(acc.at[pl.ds(s * WROWS, WROWS)], out.at[c, pl.ds(s * WROWS, WROWS)])


def _sc_segsum_l1(table, src_idx, dst_idx, zer):
    W = 48
    return pl.kernel(
        _sc_segsum_body,
        out_type=jax.ShapeDtypeStruct((NC, N, W), jnp.float32),
        mesh=plsc.VectorSubcoreMesh(core_axis_name="c", subcore_axis_name="s"),
        scratch_types=[
            pltpu.VMEM((CPB, CH), jnp.int32),
            pltpu.VMEM((CPB, CH), jnp.int32),
            pltpu.VMEM((CH, W), jnp.float32),
            pltpu.VMEM((CH, W), jnp.float32),
            pltpu.VMEM((CH, W), jnp.float32),
            pltpu.VMEM((CH, W), jnp.float32),
            pltpu.VMEM_SHARED((N + NPAD, W), jnp.float32),
            pltpu.SemaphoreType.DMA,
            pltpu.SemaphoreType.DMA,
            pltpu.SemaphoreType.DMA,
            pltpu.SemaphoreType.DMA,
            pltpu.SemaphoreType.DMA,
            pltpu.SemaphoreType.DMA,
            pltpu.SemaphoreType.DMA,
            pltpu.SemaphoreType.DMA,
        ],
        compiler_params=pltpu.CompilerParams(use_tc_tiling_on_sc=False),
    )(table, src_idx, dst_idx, zer)


def _sc_segsum2_body(table, src_idx, dst_idx, zer, out, src_v, dst_v,
                     r0, r1, r2, r3, acc, g0, g1, g2, g3, s0, s1, s2, s3):
    c = lax.axis_index("c")
    s = lax.axis_index("s")
    for p in range(2):
        q = 2 * c + p
        pltpu.sync_copy(zer, acc.at[pl.ds(s * ZROWS, ZROWS)])
        plsc.subcore_barrier()
        _sc_edge_blocks(table, src_idx.at[q], dst_idx, acc, src_v, dst_v,
                        (r0, r1, r2, r3), (g0, g1, g2, g3), (s0, s1, s2, s3))
        plsc.subcore_barrier()
        pltpu.sync_copy(acc.at[pl.ds(s * WROWS, WROWS)], out.at[q, pl.ds(s * WROWS, WROWS)])
        plsc.subcore_barrier()


def _sc_segsum_l2(table, src_idx, dst_idx, zer):
    W = 32
    return pl.kernel(
        _sc_segsum2_body,
        out_type=jax.ShapeDtypeStruct((4, N, W), jnp.float32),
        mesh=plsc.VectorSubcoreMesh(core_axis_name="c", subcore_axis_name="s"),
        scratch_types=[
            pltpu.VMEM((CPB, CH), jnp.int32),
            pltpu.VMEM((CPB, CH), jnp.int32),
            pltpu.VMEM((CH, W), jnp.float32),
            pltpu.VMEM((CH, W), jnp.float32),
            pltpu.VMEM((CH, W), jnp.float32),
            pltpu.VMEM((CH, W), jnp.float32),
            pltpu.VMEM_SHARED((N + NPAD, W), jnp.float32),
            pltpu.SemaphoreType.DMA,
            pltpu.SemaphoreType.DMA,
            pltpu.SemaphoreType.DMA,
            pltpu.SemaphoreType.DMA,
            pltpu.SemaphoreType.DMA,
            pltpu.SemaphoreType.DMA,
            pltpu.SemaphoreType.DMA,
            pltpu.SemaphoreType.DMA,
        ],
        compiler_params=pltpu.CompilerParams(use_tc_tiling_on_sc=False),
    )(table, src_idx, dst_idx, zer)


def _tc_h1_body(o0, o1, x, wl, wr, bl, h1_out, h1q_out):
    deg = o0[:, 42:43]
    rd = 1.0 / jnp.maximum(deg, 1.0)
    a0 = o0[:, :42] * rd
    a1 = o1[:, :42] * rd
    h1 = (
        jnp.dot(a0, wl[:42], preferred_element_type=jnp.float32)
        + jnp.dot(a1, wl[42:84], preferred_element_type=jnp.float32)
        + jnp.dot(x[...], wr[...], preferred_element_type=jnp.float32)
        + bl[...]
    )
    h1_out[...] = h1
    h1q_out[...] = jnp.transpose(h1.reshape(h1.shape[0], 4, 32), (1, 0, 2))


def _tc_h1(o0, o1, x, wl, wr, bl):
    blk = NODES_PER_GRAPH
    grid = (N // blk,)
    return pl.pallas_call(
        _tc_h1_body,
        grid=grid,
        in_specs=[
            pl.BlockSpec((blk, 48), lambda i: (i, 0)),
            pl.BlockSpec((blk, 48), lambda i: (i, 0)),
            pl.BlockSpec((blk, 84), lambda i: (i, 0)),
            pl.BlockSpec((84, 128), lambda i: (0, 0)),
            pl.BlockSpec((84, 128), lambda i: (0, 0)),
            pl.BlockSpec((1, 128), lambda i: (0, 0)),
        ],
        out_specs=[
            pl.BlockSpec((blk, 128), lambda i: (i, 0)),
            pl.BlockSpec((4, blk, 32), lambda i: (0, i, 0)),
        ],
        out_shape=[
            jax.ShapeDtypeStruct((N, 128), jnp.float32),
            jax.ShapeDtypeStruct((4, N, 32), jnp.float32),
        ],
    )(o0, o1, x, wl, wr, bl)


def _tc_att_body(o2, o0, h1, pe, wl, wr, bl, aggr_out):
    agg2 = jnp.concatenate([o2[0], o2[1], o2[2], o2[3]], axis=-1)
    deg = o0[:, 42:43]
    rd = 1.0 / jnp.maximum(deg, 1.0)
    h1v = h1[...]
    h2 = (
        jnp.dot(agg2 * rd, wl[...], preferred_element_type=jnp.float32)
        + jnp.dot(h1v, wr[...], preferred_element_type=jnp.float32)
        + bl[...]
    )
    t2 = h2 + pe[...]
    t2b = t2.astype(jnp.bfloat16)
    score = lax.dot_general(t2b, t2b, (((1,), (1,)), ((), ())),
                            preferred_element_type=jnp.float32) * (1.0 / math.sqrt(128.0))
    m = jnp.max(score, axis=-1, keepdims=True)
    p = jnp.exp(score - m)
    denom = jnp.sum(p, axis=-1, keepdims=True)
    ctx = jnp.dot(p.astype(jnp.bfloat16), t2b,
                  preferred_element_type=jnp.float32) / denom

    parts = []
    for tag in range(2):
        srcm = h1v if tag == 0 else ctx
        maxs = []
        means = []
        for j in range(12):
            blkv = srcm[84 * j:84 * (j + 1), :]
            maxs.append(jnp.max(blkv, axis=0, keepdims=True))
            means.append(jnp.mean(blkv, axis=0, keepdims=True))
        parts.append((jnp.concatenate(maxs, axis=0), jnp.concatenate(means, axis=0)))
    combined = jnp.concatenate([parts[0][0], parts[0][1], parts[1][0], parts[1][1]], axis=-1)
    aggr_out[...] = combined.reshape(1, 1, 6144)


def _tc_att(o2, o0, h1, pe, wl, wr, bl):
    blk = NODES_PER_GRAPH
    return pl.pallas_call(
        _tc_att_body,
        grid=(N_GRAPHS,),
        in_specs=[
            pl.BlockSpec((4, blk, 32), lambda g: (0, g, 0)),
            pl.BlockSpec((blk, 48), lambda g: (g, 0)),
            pl.BlockSpec((blk, 128), lambda g: (g, 0)),
            pl.BlockSpec((blk, 128), lambda g: (0, 0)),
            pl.BlockSpec((128, 128), lambda g: (0, 0)),
            pl.BlockSpec((128, 128), lambda g: (0, 0)),
            pl.BlockSpec((1, 128), lambda g: (0, 0)),
        ],
        out_specs=pl.BlockSpec((1, 1, 6144), lambda g: (g, 0, 0)),
        out_shape=jax.ShapeDtypeStruct((N_GRAPHS, 1, 6144), jnp.float32),
    )(o2, o0, h1, pe, wl, wr, bl)


def _tc_mlp_body(aggr, w1, b1, g1, be1, w2, b2, g2, be2, w3, b3, out):
    def bn(z, g, b):
        mm = jnp.mean(z, axis=0, keepdims=True)
        v = jnp.mean((z - mm) ** 2, axis=0, keepdims=True)
        return (z - mm) / jnp.sqrt(v + 1e-5) * g + b

    def silu(z):
        return z / (1.0 + jnp.exp(-z))

    z = silu(jnp.dot(aggr[...], w1[...], preferred_element_type=jnp.float32) + b1[...])
    z = bn(z, g1[...], be1[...])
    z = silu(jnp.dot(z, w2[...], preferred_element_type=jnp.float32) + b2[...])
    z = bn(z, g2[...], be2[...])
    logits = jnp.dot(z, w3[...], preferred_element_type=jnp.float32) + b3[...]
    l2 = logits[:, :2]
    lm = jnp.max(l2, axis=1, keepdims=True)
    e = jnp.exp(l2 - lm)
    sm = e / jnp.sum(e, axis=1, keepdims=True)
    out[...] = jnp.concatenate([sm, jnp.zeros((sm.shape[0], 126), jnp.float32)], axis=1)


def _tc_mlp(aggr, w1, b1, g1, be1, w2, b2, g2, be2, w3, b3):
    return pl.pallas_call(
        _tc_mlp_body,
        out_shape=jax.ShapeDtypeStruct((N_GRAPHS, 128), jnp.float32),
    )(aggr, w1, b1, g1, be1, w2, b2, g2, be2, w3, b3)


def kernel(x, edge_index, pe, Wl1, bl1, Wr1, Wl2, bl2, Wr2, W_fc1, b_fc1, g1,
           be1, W_fc2, b_fc2, g2, be2, W_fc3, b_fc3):
    f32 = jnp.float32
    src = edge_index[0]
    dst = edge_index[1]
    npad = EPAD - E
    src_pad = (jnp.arange(npad, dtype=jnp.int32) * 63) % N
    dst_pad = N + (jnp.arange(npad, dtype=jnp.int32) % NPAD)
    srcr = jnp.concatenate([src, src_pad]).reshape(NS, NCH, CH)
    dstr = jnp.concatenate([dst, dst_pad]).reshape(NS, NCH, CH)
    src_l1 = jnp.stack([srcr, srcr + N])
    src_l2 = jnp.stack([srcr + q * N for q in range(4)])

    ones = jnp.ones((N, 1), f32)
    t0 = jnp.concatenate([x[:, :42], ones, jnp.zeros((N, 5), f32)], axis=1)
    t1 = jnp.concatenate([x[:, 42:84], jnp.zeros((N, 6), f32)], axis=1)
    table1 = jnp.concatenate([t0, t1], axis=0)
    zer48 = jnp.zeros((ZROWS, 48), f32)
    zer32 = jnp.zeros((ZROWS, 32), f32)

    oA = _sc_segsum_l1(table1, src_l1, dstr, zer48)

    h1, h1q = _tc_h1(oA[0], oA[1], x, Wl1, Wr1, bl1.reshape(1, 128))

    o2 = _sc_segsum_l2(h1q.reshape(4 * N, 32), src_l2, dstr, zer32)

    aggr = _tc_att(o2, oA[0], h1, pe, Wl2, Wr2, bl2.reshape(1, 128)).reshape(N_GRAPHS, 6144)

    w2p = jnp.pad(W_fc2, ((0, 0), (0, 96)))
    b2p = jnp.pad(b_fc2, (0, 96)).reshape(1, 128)
    g2p = jnp.pad(g2, (0, 96)).reshape(1, 128)
    be2p = jnp.pad(be2, (0, 96)).reshape(1, 128)
    w3p = jnp.pad(W_fc3, ((0, 96), (0, 126)))
    b3p = jnp.pad(b_fc3, (0, 126)).reshape(1, 128)
    outp = _tc_mlp(aggr, W_fc1, b_fc1.reshape(1, 512), g1.reshape(1, 512),
                   be1.reshape(1, 512), w2p, b2p, g2p, be2p, w3p, b3p)
    return outp[:, :2]


# speedup vs baseline: 11.4820x; 1.3097x over previous
import functools
import math

import jax
import jax.numpy as jnp
from jax import lax
from jax.experimental import pallas as pl
from jax.experimental.pallas import tpu as pltpu
from jax.experimental.pallas import tpu_sc as plsc

N_GRAPHS = 32
NODES_PER_GRAPH = 1008
N = N_GRAPHS * NODES_PER_GRAPH
E = N * 16
NC, NS = 2, 16
EPAD = 524288
CH = 128
NCH = EPAD // NS // CH
NPAD = 384
CPB = 32
NIB = NCH // CPB
NIB_GROUPS = CPB // 4
ZROWS = (N + NPAD) // NS
WROWS = N // NS


def _sc_edge_blocks(table, src_idx, dst_idx, mul, off, acc, src_v, dst_v, rows, gsem, ssem):
    s = lax.axis_index("s")

    def gath(i, t):
        pltpu.async_copy(table.at[src_v.at[i]], rows[t], gsem[t])

    def wait_g(t):
        pltpu.make_async_copy(table.at[src_v.at[0]], rows[t], gsem[t]).wait()

    def scat(i, t):
        pltpu.async_copy(rows[t], acc.at[dst_v.at[i]], ssem[t], add=True)

    def wait_s(t):
        pltpu.make_async_copy(rows[t], acc.at[dst_v.at[0]], ssem[t]).wait()

    def iblock(b, carry):
        pltpu.sync_copy(src_idx.at[s, pl.ds(b * CPB, CPB)], src_v)
        pltpu.sync_copy(dst_idx.at[s, pl.ds(b * CPB, CPB)], dst_v)

        def offrow(r, carry3):
            for j in range(CH // 16):
                sl = (r, pl.ds(j * 16, 16))
                src_v[sl] = src_v[sl] * mul + off
            return carry3

        lax.fori_loop(0, CPB, offrow, 0)
        gath(0, 0)
        gath(1, 1)
        for t in range(4):
            wait_g(t)
            scat(t, t)
            if t < 2:
                gath(t + 2, t + 2)
            else:
                wait_s(t - 2)
                gath(t + 2, t - 2)

        def group(j, carry2):
            c = 4 * j
            for t in range(4):
                wait_g(t)
                scat(c + t, t)
                wait_s((t + 2) % 4)
                gath(c + t + 2, (t + 2) % 4)
            return carry2

        lax.fori_loop(1, NIB_GROUPS - 1, group, carry)
        c = CPB - 4
        for t in range(4):
            wait_g(t)
            scat(c + t, t)
            if t < 2:
                wait_s(t + 2)
                gath(c + t + 2, t + 2)
        for t in range(4):
            wait_s(t)
        return carry

    lax.fori_loop(0, NIB, iblock, 0)


def _make_sc_segsum_body(mul):
    def body(table, src_idx, dst_idx, zer, out, src_v, dst_v,
             r0, r1, r2, r3, acc, g0, g1, g2, g3, s0, s1, s2, s3):
        c = lax.axis_index("c")
        s = lax.axis_index("s")
        off = c * N if mul == 1 else c
        pltpu.sync_copy(zer, acc.at[pl.ds(s * ZROWS, ZROWS)])
        plsc.subcore_barrier()
        _sc_edge_blocks(table, src_idx, dst_idx, mul, off, acc, src_v, dst_v,
                        (r0, r1, r2, r3), (g0, g1, g2, g3), (s0, s1, s2, s3))
        plsc.subcore_barrier()
        pltpu.sync_copy(acc.at[pl.ds(s * WROWS, WROWS)], out.at[c, pl.ds(s * WROWS, WROWS)])

    return body


def _sc_segsum_l1(table, src_idx, dst_idx, zer):
    W = 48
    return pl.kernel(
        _make_sc_segsum_body(1),
        out_type=jax.ShapeDtypeStruct((NC, N, W), jnp.bfloat16),
        mesh=plsc.VectorSubcoreMesh(core_axis_name="c", subcore_axis_name="s"),
        scratch_types=[
            pltpu.VMEM((CPB, CH), jnp.int32),
            pltpu.VMEM((CPB, CH), jnp.int32),
            pltpu.VMEM((CH, W), jnp.bfloat16),
            pltpu.VMEM((CH, W), jnp.bfloat16),
            pltpu.VMEM((CH, W), jnp.bfloat16),
            pltpu.VMEM((CH, W), jnp.bfloat16),
            pltpu.VMEM_SHARED((N + NPAD, W), jnp.bfloat16),
            pltpu.SemaphoreType.DMA,
            pltpu.SemaphoreType.DMA,
            pltpu.SemaphoreType.DMA,
            pltpu.SemaphoreType.DMA,
            pltpu.SemaphoreType.DMA,
            pltpu.SemaphoreType.DMA,
            pltpu.SemaphoreType.DMA,
            pltpu.SemaphoreType.DMA,
        ],
        compiler_params=pltpu.CompilerParams(use_tc_tiling_on_sc=False),
    )(table, src_idx, dst_idx, zer)


def _sc_segsum_l2(table, src_idx, dst_idx, zer):
    W = 64
    return pl.kernel(
        _make_sc_segsum_body(2),
        out_type=jax.ShapeDtypeStruct((NC, N, W), jnp.bfloat16),
        mesh=plsc.VectorSubcoreMesh(core_axis_name="c", subcore_axis_name="s"),
        scratch_types=[
            pltpu.VMEM((CPB, CH), jnp.int32),
            pltpu.VMEM((CPB, CH), jnp.int32),
            pltpu.VMEM((CH, W), jnp.bfloat16),
            pltpu.VMEM((CH, W), jnp.bfloat16),
            pltpu.VMEM((CH, W), jnp.bfloat16),
            pltpu.VMEM((CH, W), jnp.bfloat16),
            pltpu.VMEM_SHARED((N + NPAD, W), jnp.bfloat16),
            pltpu.SemaphoreType.DMA,
            pltpu.SemaphoreType.DMA,
            pltpu.SemaphoreType.DMA,
            pltpu.SemaphoreType.DMA,
            pltpu.SemaphoreType.DMA,
            pltpu.SemaphoreType.DMA,
            pltpu.SemaphoreType.DMA,
            pltpu.SemaphoreType.DMA,
        ],
        compiler_params=pltpu.CompilerParams(use_tc_tiling_on_sc=False),
    )(table, src_idx, dst_idx, zer)


def _tc_h1_body(oa, x, wl, wr, bl, h1_out, h1h_out):
    o0 = oa[0]
    o1 = oa[1]
    deg = o0[:, 42:43].astype(jnp.float32)
    rd = 1.0 / jnp.maximum(deg, 1.0)
    a0 = o0[:, :42].astype(jnp.float32) * rd
    a1 = o1[:, :42].astype(jnp.float32) * rd
    h1 = (
        jnp.dot(a0, wl[:42], preferred_element_type=jnp.float32)
        + jnp.dot(a1, wl[42:84], preferred_element_type=jnp.float32)
        + jnp.dot(x[...], wr[...], preferred_element_type=jnp.float32)
        + bl[...]
    )
    h1_out[...] = h1
    h1h_out[...] = h1.astype(jnp.bfloat16)


def _tc_h1(oa, x, wl, wr, bl):
    blk = NODES_PER_GRAPH
    grid = (N // blk,)
    return pl.pallas_call(
        _tc_h1_body,
        grid=grid,
        in_specs=[
            pl.BlockSpec((2, blk, 48), lambda i: (0, i, 0)),
            pl.BlockSpec((blk, 84), lambda i: (i, 0)),
            pl.BlockSpec((84, 128), lambda i: (0, 0)),
            pl.BlockSpec((84, 128), lambda i: (0, 0)),
            pl.BlockSpec((1, 128), lambda i: (0, 0)),
        ],
        out_specs=[
            pl.BlockSpec((blk, 128), lambda i: (i, 0)),
            pl.BlockSpec((blk, 128), lambda i: (i, 0)),
        ],
        out_shape=[
            jax.ShapeDtypeStruct((N, 128), jnp.float32),
            jax.ShapeDtypeStruct((N, 128), jnp.bfloat16),
        ],
    )(oa, x, wl, wr, bl)


def _tc_att_body(o2, oa, h1, pe, wl, wr, bl, aggr_out):
    agg2 = jnp.concatenate([o2[0], o2[1]], axis=-1).astype(jnp.float32)
    deg = oa[0, :, 42:43].astype(jnp.float32)
    rd = 1.0 / jnp.maximum(deg, 1.0)
    h1v = h1[...]
    h2 = (
        jnp.dot(agg2 * rd, wl[...], preferred_element_type=jnp.float32)
        + jnp.dot(h1v, wr[...], preferred_element_type=jnp.float32)
        + bl[...]
    )
    t2 = h2 + pe[...]
    t2b = t2.astype(jnp.bfloat16)
    score = lax.dot_general(t2b, t2b, (((1,), (1,)), ((), ())),
                            preferred_element_type=jnp.float32) * (1.0 / math.sqrt(128.0))
    m = jnp.max(score, axis=-1, keepdims=True)
    p = jnp.exp(score - m)
    denom = jnp.sum(p, axis=-1, keepdims=True)
    ctx = jnp.dot(p.astype(jnp.bfloat16), t2b,
                  preferred_element_type=jnp.float32) / denom

    parts = []
    for tag in range(2):
        srcm = h1v if tag == 0 else ctx
        maxs = []
        means = []
        for j in range(12):
            blkv = srcm[84 * j:84 * (j + 1), :]
            maxs.append(jnp.max(blkv, axis=0, keepdims=True))
            means.append(jnp.mean(blkv, axis=0, keepdims=True))
        parts.append((jnp.concatenate(maxs, axis=0), jnp.concatenate(means, axis=0)))
    combined = jnp.concatenate([parts[0][0], parts[0][1], parts[1][0], parts[1][1]], axis=-1)
    aggr_out[...] = combined.reshape(1, 1, 6144)


def _tc_att(o2, o0, h1, pe, wl, wr, bl):
    blk = NODES_PER_GRAPH
    return pl.pallas_call(
        _tc_att_body,
        grid=(N_GRAPHS,),
        in_specs=[
            pl.BlockSpec((2, blk, 64), lambda g: (0, g, 0)),
            pl.BlockSpec((1, blk, 48), lambda g: (0, g, 0)),
            pl.BlockSpec((blk, 128), lambda g: (g, 0)),
            pl.BlockSpec((blk, 128), lambda g: (0, 0)),
            pl.BlockSpec((128, 128), lambda g: (0, 0)),
            pl.BlockSpec((128, 128), lambda g: (0, 0)),
            pl.BlockSpec((1, 128), lambda g: (0, 0)),
        ],
        out_specs=pl.BlockSpec((1, 1, 6144), lambda g: (g, 0, 0)),
        out_shape=jax.ShapeDtypeStruct((N_GRAPHS, 1, 6144), jnp.float32),
    )(o2, o0, h1, pe, wl, wr, bl)


def _tc_mlp_body(aggr, w1, b1, g1, be1, w2, b2, g2, be2, w3, b3, out):
    def bn(z, g, b):
        mm = jnp.mean(z, axis=0, keepdims=True)
        v = jnp.mean((z - mm) ** 2, axis=0, keepdims=True)
        return (z - mm) / jnp.sqrt(v + 1e-5) * g + b

    def silu(z):
        return z / (1.0 + jnp.exp(-z))

    z = silu(jnp.dot(aggr[...], w1[...], preferred_element_type=jnp.float32) + b1[...])
    z = bn(z, g1[...], be1[...])
    z = silu(jnp.dot(z, w2[...], preferred_element_type=jnp.float32) + b2[...])
    z = bn(z, g2[...], be2[...])
    logits = jnp.dot(z, w3[...], preferred_element_type=jnp.float32) + b3[...]
    l2 = logits[:, :2]
    lm = jnp.max(l2, axis=1, keepdims=True)
    e = jnp.exp(l2 - lm)
    sm = e / jnp.sum(e, axis=1, keepdims=True)
    out[...] = jnp.concatenate([sm, jnp.zeros((sm.shape[0], 126), jnp.float32)], axis=1)


def _tc_mlp(aggr, w1, b1, g1, be1, w2, b2, g2, be2, w3, b3):
    return pl.pallas_call(
        _tc_mlp_body,
        out_shape=jax.ShapeDtypeStruct((N_GRAPHS, 128), jnp.float32),
    )(aggr, w1, b1, g1, be1, w2, b2, g2, be2, w3, b3)


def kernel(x, edge_index, pe, Wl1, bl1, Wr1, Wl2, bl2, Wr2, W_fc1, b_fc1, g1,
           be1, W_fc2, b_fc2, g2, be2, W_fc3, b_fc3):
    f32 = jnp.float32
    src = edge_index[0]
    dst = edge_index[1]
    npad = EPAD - E
    src_pad = (jnp.arange(npad, dtype=jnp.int32) * 63) % N
    dst_pad = N + (jnp.arange(npad, dtype=jnp.int32) % NPAD)
    bf16 = jnp.bfloat16
    srcr = jnp.concatenate([src, src_pad]).reshape(NS, NCH, CH)
    dstr = jnp.concatenate([dst, dst_pad]).reshape(NS, NCH, CH)

    ones = jnp.ones((N, 1), f32)
    t0 = jnp.concatenate([x[:, :42], ones, jnp.zeros((N, 5), f32)], axis=1).astype(bf16)
    t1 = jnp.concatenate([x[:, 42:84], jnp.zeros((N, 6), f32)], axis=1).astype(bf16)
    table1 = jnp.concatenate([t0, t1], axis=0)
    zer48 = jnp.zeros((ZROWS, 48), bf16)
    zer64 = jnp.zeros((ZROWS, 64), bf16)

    oA = _sc_segsum_l1(table1, srcr, dstr, zer48)

    h1, h1b = _tc_h1(oA, x, Wl1, Wr1, bl1.reshape(1, 128))

    o2 = _sc_segsum_l2(h1b.reshape(2 * N, 64), srcr, dstr, zer64)

    aggr = _tc_att(o2, oA, h1, pe, Wl2, Wr2, bl2.reshape(1, 128)).reshape(N_GRAPHS, 6144)

    w2p = jnp.pad(W_fc2, ((0, 0), (0, 96)))
    b2p = jnp.pad(b_fc2, (0, 96)).reshape(1, 128)
    g2p = jnp.pad(g2, (0, 96)).reshape(1, 128)
    be2p = jnp.pad(be2, (0, 96)).reshape(1, 128)
    w3p = jnp.pad(W_fc3, ((0, 96), (0, 126)))
    b3p = jnp.pad(b_fc3, (0, 126)).reshape(1, 128)
    outp = _tc_mlp(aggr, W_fc1, b_fc1.reshape(1, 512), g1.reshape(1, 512),
                   be1.reshape(1, 512), w2p, b2p, g2p, be2p, w3p, b3p)
    return outp[:, :2]


# speedup vs baseline: 12.6371x; 1.1006x over previous
import functools
import math

import jax
import jax.numpy as jnp
from jax import lax
from jax.experimental import pallas as pl
from jax.experimental.pallas import tpu as pltpu
from jax.experimental.pallas import tpu_sc as plsc

N_GRAPHS = 32
NODES_PER_GRAPH = 1008
N = N_GRAPHS * NODES_PER_GRAPH
E = N * 16
NC, NS = 2, 16
EPAD = 524288
CH = 128
NCH = EPAD // NS // CH
NPAD = 256
CPB = 32
NIB = NCH // CPB
NIB_GROUPS = CPB // 4
ZROWS = (N + NPAD) // NS
WROWS = N // NS


def _sc_edge_blocks(table, src_idx, dst_idx, mul, off, blo, bhi, acc, src_v, dst_v, rows, gsem, ssem):
    s = lax.axis_index("s")

    def gath(i, t):
        pltpu.async_copy(table.at[src_v.at[i]], rows[t], gsem[t])

    def wait_g(t):
        pltpu.make_async_copy(table.at[src_v.at[0]], rows[t], gsem[t]).wait()

    def scat(i, t):
        pltpu.async_copy(rows[t], acc.at[dst_v.at[i]], ssem[t], add=True)

    def wait_s(t):
        pltpu.make_async_copy(rows[t], acc.at[dst_v.at[0]], ssem[t]).wait()

    def iblock(b, carry):
        pltpu.sync_copy(src_idx.at[s, pl.ds(b * CPB, CPB)], src_v)
        pltpu.sync_copy(dst_idx.at[s, pl.ds(b * CPB, CPB)], dst_v)

        if mul != 1 or off is not None:
            def offrow(r, carry3):
                for j in range(CH // 16):
                    sl = (r, pl.ds(j * 16, 16))
                    src_v[sl] = src_v[sl] * mul + off
                return carry3

            lax.fori_loop(0, CPB, offrow, 0)
        gath(0, 0)
        gath(1, 1)
        for t in range(4):
            wait_g(t)
            scat(t, t)
            if t < 2:
                gath(t + 2, t + 2)
            else:
                wait_s(t - 2)
                gath(t + 2, t - 2)

        def group(j, carry2):
            c = 4 * j
            for t in range(4):
                wait_g(t)
                scat(c + t, t)
                wait_s((t + 2) % 4)
                gath(c + t + 2, (t + 2) % 4)
            return carry2

        lax.fori_loop(1, NIB_GROUPS - 1, group, carry)
        c = CPB - 4
        for t in range(4):
            wait_g(t)
            scat(c + t, t)
            if t < 2:
                wait_s(t + 2)
                gath(c + t + 2, t + 2)
        for t in range(4):
            wait_s(t)
        return carry

    lax.fori_loop(blo, bhi, iblock, 0)


def _make_sc_segsum_body(mode):
    def body(table, src_idx, dst_idx, zer, out, src_v, dst_v,
             r0, r1, r2, r3, acc, g0, g1, g2, g3, s0, s1, s2, s3):
        c = lax.axis_index("c")
        s = lax.axis_index("s")
        if mode == "edge_split":
            mul, off = 1, None
            blo, bhi = c * (NIB // 2), (c + 1) * (NIB // 2)
        else:
            mul, off = 2, c
            blo, bhi = 0, NIB
        pltpu.sync_copy(zer, acc.at[pl.ds(s * ZROWS, ZROWS)])
        plsc.subcore_barrier()
        _sc_edge_blocks(table, src_idx, dst_idx, mul, off, blo, bhi, acc, src_v, dst_v,
                        (r0, r1, r2, r3), (g0, g1, g2, g3), (s0, s1, s2, s3))
        plsc.subcore_barrier()
        pltpu.sync_copy(acc.at[pl.ds(s * WROWS, WROWS)], out.at[c, pl.ds(s * WROWS, WROWS)])

    return body


def _sc_segsum_l1(table, src_idx, dst_idx, zer):
    W = 96
    return pl.kernel(
        _make_sc_segsum_body("edge_split"),
        out_type=jax.ShapeDtypeStruct((NC, N, W), jnp.bfloat16),
        mesh=plsc.VectorSubcoreMesh(core_axis_name="c", subcore_axis_name="s"),
        scratch_types=[
            pltpu.VMEM((CPB, CH), jnp.int32),
            pltpu.VMEM((CPB, CH), jnp.int32),
            pltpu.VMEM((CH, W), jnp.bfloat16),
            pltpu.VMEM((CH, W), jnp.bfloat16),
            pltpu.VMEM((CH, W), jnp.bfloat16),
            pltpu.VMEM((CH, W), jnp.bfloat16),
            pltpu.VMEM_SHARED((N + NPAD, W), jnp.bfloat16),
            pltpu.SemaphoreType.DMA,
            pltpu.SemaphoreType.DMA,
            pltpu.SemaphoreType.DMA,
            pltpu.SemaphoreType.DMA,
            pltpu.SemaphoreType.DMA,
            pltpu.SemaphoreType.DMA,
            pltpu.SemaphoreType.DMA,
            pltpu.SemaphoreType.DMA,
        ],
        compiler_params=pltpu.CompilerParams(use_tc_tiling_on_sc=False),
    )(table, src_idx, dst_idx, zer)


def _sc_segsum_l2(table, src_idx, dst_idx, zer):
    W = 64
    return pl.kernel(
        _make_sc_segsum_body("interleave"),
        out_type=jax.ShapeDtypeStruct((NC, N, W), jnp.bfloat16),
        mesh=plsc.VectorSubcoreMesh(core_axis_name="c", subcore_axis_name="s"),
        scratch_types=[
            pltpu.VMEM((CPB, CH), jnp.int32),
            pltpu.VMEM((CPB, CH), jnp.int32),
            pltpu.VMEM((CH, W), jnp.bfloat16),
            pltpu.VMEM((CH, W), jnp.bfloat16),
            pltpu.VMEM((CH, W), jnp.bfloat16),
            pltpu.VMEM((CH, W), jnp.bfloat16),
            pltpu.VMEM_SHARED((N + NPAD, W), jnp.bfloat16),
            pltpu.SemaphoreType.DMA,
            pltpu.SemaphoreType.DMA,
            pltpu.SemaphoreType.DMA,
            pltpu.SemaphoreType.DMA,
            pltpu.SemaphoreType.DMA,
            pltpu.SemaphoreType.DMA,
            pltpu.SemaphoreType.DMA,
            pltpu.SemaphoreType.DMA,
        ],
        compiler_params=pltpu.CompilerParams(use_tc_tiling_on_sc=False),
    )(table, src_idx, dst_idx, zer)


def _tc_h1_body(oa, x, wl, wr, bl, h1_out, h1h_out):
    o = oa[0].astype(jnp.float32) + oa[1].astype(jnp.float32)
    deg = o[:, 84:85]
    rd = 1.0 / jnp.maximum(deg, 1.0)
    a = o[:, :84] * rd
    h1 = (
        jnp.dot(a, wl[...], preferred_element_type=jnp.float32)
        + jnp.dot(x[...], wr[...], preferred_element_type=jnp.float32)
        + bl[...]
    )
    h1_out[...] = h1
    h1h_out[...] = h1.astype(jnp.bfloat16)


def _tc_h1(oa, x, wl, wr, bl):
    blk = NODES_PER_GRAPH
    grid = (N // blk,)
    return pl.pallas_call(
        _tc_h1_body,
        grid=grid,
        in_specs=[
            pl.BlockSpec((2, blk, 96), lambda i: (0, i, 0)),
            pl.BlockSpec((blk, 84), lambda i: (i, 0)),
            pl.BlockSpec((84, 128), lambda i: (0, 0)),
            pl.BlockSpec((84, 128), lambda i: (0, 0)),
            pl.BlockSpec((1, 128), lambda i: (0, 0)),
        ],
        out_specs=[
            pl.BlockSpec((blk, 128), lambda i: (i, 0)),
            pl.BlockSpec((blk, 128), lambda i: (i, 0)),
        ],
        out_shape=[
            jax.ShapeDtypeStruct((N, 128), jnp.float32),
            jax.ShapeDtypeStruct((N, 128), jnp.bfloat16),
        ],
    )(oa, x, wl, wr, bl)


def _tc_att_body(o2, oa, h1, pe, wl, wr, bl, aggr_out):
    agg2 = jnp.concatenate([o2[0], o2[1]], axis=-1).astype(jnp.float32)
    deg = (oa[0, :, 84:85].astype(jnp.float32)
           + oa[1, :, 84:85].astype(jnp.float32))
    rd = 1.0 / jnp.maximum(deg, 1.0)
    h1v = h1[...]
    h2 = (
        jnp.dot(agg2 * rd, wl[...], preferred_element_type=jnp.float32)
        + jnp.dot(h1v, wr[...], preferred_element_type=jnp.float32)
        + bl[...]
    )
    t2 = h2 + pe[...]
    t2b = t2.astype(jnp.bfloat16)
    score = lax.dot_general(t2b, t2b, (((1,), (1,)), ((), ())),
                            preferred_element_type=jnp.float32) * (1.0 / math.sqrt(128.0))
    p = jnp.exp(score)
    denom = jnp.sum(p, axis=-1, keepdims=True)
    ctx = jnp.dot(p.astype(jnp.bfloat16), t2b,
                  preferred_element_type=jnp.float32) / denom

    parts = []
    for tag in range(2):
        srcm = h1v if tag == 0 else ctx
        maxs = []
        means = []
        for j in range(12):
            blkv = srcm[84 * j:84 * (j + 1), :]
            maxs.append(jnp.max(blkv, axis=0, keepdims=True))
            means.append(jnp.mean(blkv, axis=0, keepdims=True))
        parts.append((jnp.concatenate(maxs, axis=0), jnp.concatenate(means, axis=0)))
    combined = jnp.concatenate([parts[0][0], parts[0][1], parts[1][0], parts[1][1]], axis=-1)
    aggr_out[...] = combined.reshape(1, 1, 6144)


def _tc_att(o2, o0, h1, pe, wl, wr, bl):
    blk = NODES_PER_GRAPH
    return pl.pallas_call(
        _tc_att_body,
        grid=(N_GRAPHS,),
        in_specs=[
            pl.BlockSpec((2, blk, 64), lambda g: (0, g, 0)),
            pl.BlockSpec((2, blk, 96), lambda g: (0, g, 0)),
            pl.BlockSpec((blk, 128), lambda g: (g, 0)),
            pl.BlockSpec((blk, 128), lambda g: (0, 0)),
            pl.BlockSpec((128, 128), lambda g: (0, 0)),
            pl.BlockSpec((128, 128), lambda g: (0, 0)),
            pl.BlockSpec((1, 128), lambda g: (0, 0)),
        ],
        out_specs=pl.BlockSpec((1, 1, 6144), lambda g: (g, 0, 0)),
        out_shape=jax.ShapeDtypeStruct((N_GRAPHS, 1, 6144), jnp.float32),
    )(o2, o0, h1, pe, wl, wr, bl)


def _tc_mlp_body(aggr, w1, b1, g1, be1, w2, b2, g2, be2, w3, b3, out):
    def bn(z, g, b):
        mm = jnp.mean(z, axis=0, keepdims=True)
        v = jnp.mean((z - mm) ** 2, axis=0, keepdims=True)
        return (z - mm) / jnp.sqrt(v + 1e-5) * g + b

    def silu(z):
        return z / (1.0 + jnp.exp(-z))

    z = silu(jnp.dot(aggr[...], w1[...], preferred_element_type=jnp.float32) + b1[...])
    z = bn(z, g1[...], be1[...])
    z = silu(jnp.dot(z, w2[...], preferred_element_type=jnp.float32) + b2[...])
    z = bn(z, g2[...], be2[...])
    logits = jnp.dot(z, w3[...], preferred_element_type=jnp.float32) + b3[...]
    l2 = logits[:, :2]
    lm = jnp.max(l2, axis=1, keepdims=True)
    e = jnp.exp(l2 - lm)
    sm = e / jnp.sum(e, axis=1, keepdims=True)
    out[...] = jnp.concatenate([sm, jnp.zeros((sm.shape[0], 126), jnp.float32)], axis=1)


def _tc_mlp(aggr, w1, b1, g1, be1, w2, b2, g2, be2, w3, b3):
    return pl.pallas_call(
        _tc_mlp_body,
        out_shape=jax.ShapeDtypeStruct((N_GRAPHS, 128), jnp.float32),
    )(aggr, w1, b1, g1, be1, w2, b2, g2, be2, w3, b3)


def kernel(x, edge_index, pe, Wl1, bl1, Wr1, Wl2, bl2, Wr2, W_fc1, b_fc1, g1,
           be1, W_fc2, b_fc2, g2, be2, W_fc3, b_fc3):
    f32 = jnp.float32
    src = edge_index[0]
    dst = edge_index[1]
    npad = EPAD - E
    src_pad = (jnp.arange(npad, dtype=jnp.int32) * 63) % N
    dst_pad = N + (jnp.arange(npad, dtype=jnp.int32) % NPAD)
    bf16 = jnp.bfloat16
    srcr = jnp.concatenate([src, src_pad]).reshape(NS, NCH, CH)
    dstr = jnp.concatenate([dst, dst_pad]).reshape(NS, NCH, CH)

    ones = jnp.ones((N, 1), f32)
    table1 = jnp.concatenate([x, ones, jnp.zeros((N, 11), f32)], axis=1).astype(bf16)
    zer96 = jnp.zeros((ZROWS, 96), bf16)
    zer64 = jnp.zeros((ZROWS, 64), bf16)

    oA = _sc_segsum_l1(table1, srcr, dstr, zer96)

    h1, h1b = _tc_h1(oA, x, Wl1, Wr1, bl1.reshape(1, 128))

    o2 = _sc_segsum_l2(h1b.reshape(2 * N, 64), srcr, dstr, zer64)

    aggr = _tc_att(o2, oA, h1, pe, Wl2, Wr2, bl2.reshape(1, 128)).reshape(N_GRAPHS, 6144)

    w2p = jnp.pad(W_fc2, ((0, 0), (0, 96)))
    b2p = jnp.pad(b_fc2, (0, 96)).reshape(1, 128)
    g2p = jnp.pad(g2, (0, 96)).reshape(1, 128)
    be2p = jnp.pad(be2, (0, 96)).reshape(1, 128)
    w3p = jnp.pad(W_fc3, ((0, 96), (0, 126)))
    b3p = jnp.pad(b_fc3, (0, 126)).reshape(1, 128)
    outp = _tc_mlp(aggr, W_fc1, b_fc1.reshape(1, 512), g1.reshape(1, 512),
                   be1.reshape(1, 512), w2p, b2p, g2p, be2p, w3p, b3p)
    return outp[:, :2]


# speedup vs baseline: 12.6555x; 1.0015x over previous
import functools
import math

import jax
import jax.numpy as jnp
from jax import lax
from jax.experimental import pallas as pl
from jax.experimental.pallas import tpu as pltpu
from jax.experimental.pallas import tpu_sc as plsc

N_GRAPHS = 32
NODES_PER_GRAPH = 1008
N = N_GRAPHS * NODES_PER_GRAPH
E = N * 16
NC, NS = 2, 16
EPAD = 524288
CH = 128
NCH = EPAD // NS // CH
NPAD = 256
CPB = 32
NIB = NCH // CPB
NIB_GROUPS = CPB // 4
ZROWS = (N + NPAD) // NS
WROWS = N // NS


def _sc_edge_blocks(table, src_idx, dst_idx, mul, off, blo, bhi, acc, src_v, dst_v, rows, gsem, ssem):
    s = lax.axis_index("s")

    def gath(i, t):
        pltpu.async_copy(table.at[src_v.at[i]], rows[t], gsem[t])

    def wait_g(t):
        pltpu.make_async_copy(table.at[src_v.at[0]], rows[t], gsem[t]).wait()

    def scat(i, t):
        pltpu.async_copy(rows[t], acc.at[dst_v.at[i]], ssem[t], add=True)

    def wait_s(t):
        pltpu.make_async_copy(rows[t], acc.at[dst_v.at[0]], ssem[t]).wait()

    def iblock(b, carry):
        pltpu.sync_copy(src_idx.at[s, pl.ds(b * CPB, CPB)], src_v)
        pltpu.sync_copy(dst_idx.at[s, pl.ds(b * CPB, CPB)], dst_v)

        if mul != 1 or off is not None:
            def offrow(r, carry3):
                for j in range(CH // 16):
                    sl = (r, pl.ds(j * 16, 16))
                    src_v[sl] = src_v[sl] * mul + off
                return carry3

            lax.fori_loop(0, CPB, offrow, 0)
        gath(0, 0)
        gath(1, 1)
        for t in range(4):
            wait_g(t)
            scat(t, t)
            if t < 2:
                gath(t + 2, t + 2)
            else:
                wait_s(t - 2)
                gath(t + 2, t - 2)

        def group(j, carry2):
            c = 4 * j
            for t in range(4):
                wait_g(t)
                scat(c + t, t)
                wait_s((t + 2) % 4)
                gath(c + t + 2, (t + 2) % 4)
            return carry2

        lax.fori_loop(1, NIB_GROUPS - 1, group, carry)
        c = CPB - 4
        for t in range(4):
            wait_g(t)
            scat(c + t, t)
            if t < 2:
                wait_s(t + 2)
                gath(c + t + 2, t + 2)
        for t in range(4):
            wait_s(t)
        return carry

    lax.fori_loop(blo, bhi, iblock, 0)


def _make_sc_segsum_body(mode):
    def body(table, src_idx, dst_idx, zer, out, src_v, dst_v,
             r0, r1, r2, r3, acc, g0, g1, g2, g3, s0, s1, s2, s3):
        c = lax.axis_index("c")
        s = lax.axis_index("s")
        if mode == "edge_split":
            mul, off = 1, None
            blo, bhi = c * (NIB // 2), (c + 1) * (NIB // 2)
        else:
            mul, off = 2, c
            blo, bhi = 0, NIB
        pltpu.sync_copy(zer, acc.at[pl.ds(s * ZROWS, ZROWS)])
        plsc.subcore_barrier()
        _sc_edge_blocks(table, src_idx, dst_idx, mul, off, blo, bhi, acc, src_v, dst_v,
                        (r0, r1, r2, r3), (g0, g1, g2, g3), (s0, s1, s2, s3))
        plsc.subcore_barrier()
        pltpu.sync_copy(acc.at[pl.ds(s * WROWS, WROWS)], out.at[c, pl.ds(s * WROWS, WROWS)])

    return body


def _sc_segsum_l1(table, src_idx, dst_idx, zer):
    W = 96
    return pl.kernel(
        _make_sc_segsum_body("edge_split"),
        out_type=jax.ShapeDtypeStruct((NC, N, W), jnp.bfloat16),
        mesh=plsc.VectorSubcoreMesh(core_axis_name="c", subcore_axis_name="s"),
        scratch_types=[
            pltpu.VMEM((CPB, CH), jnp.int32),
            pltpu.VMEM((CPB, CH), jnp.int32),
            pltpu.VMEM((CH, W), jnp.bfloat16),
            pltpu.VMEM((CH, W), jnp.bfloat16),
            pltpu.VMEM((CH, W), jnp.bfloat16),
            pltpu.VMEM((CH, W), jnp.bfloat16),
            pltpu.VMEM_SHARED((N + NPAD, W), jnp.bfloat16),
            pltpu.SemaphoreType.DMA,
            pltpu.SemaphoreType.DMA,
            pltpu.SemaphoreType.DMA,
            pltpu.SemaphoreType.DMA,
            pltpu.SemaphoreType.DMA,
            pltpu.SemaphoreType.DMA,
            pltpu.SemaphoreType.DMA,
            pltpu.SemaphoreType.DMA,
        ],
        compiler_params=pltpu.CompilerParams(use_tc_tiling_on_sc=False),
    )(table, src_idx, dst_idx, zer)


def _sc_segsum_l2(table, src_idx, dst_idx, zer):
    W = 64
    return pl.kernel(
        _make_sc_segsum_body("interleave"),
        out_type=jax.ShapeDtypeStruct((NC, N, W), jnp.bfloat16),
        mesh=plsc.VectorSubcoreMesh(core_axis_name="c", subcore_axis_name="s"),
        scratch_types=[
            pltpu.VMEM((CPB, CH), jnp.int32),
            pltpu.VMEM((CPB, CH), jnp.int32),
            pltpu.VMEM((CH, W), jnp.bfloat16),
            pltpu.VMEM((CH, W), jnp.bfloat16),
            pltpu.VMEM((CH, W), jnp.bfloat16),
            pltpu.VMEM((CH, W), jnp.bfloat16),
            pltpu.VMEM_SHARED((N + NPAD, W), jnp.bfloat16),
            pltpu.SemaphoreType.DMA,
            pltpu.SemaphoreType.DMA,
            pltpu.SemaphoreType.DMA,
            pltpu.SemaphoreType.DMA,
            pltpu.SemaphoreType.DMA,
            pltpu.SemaphoreType.DMA,
            pltpu.SemaphoreType.DMA,
            pltpu.SemaphoreType.DMA,
        ],
        compiler_params=pltpu.CompilerParams(use_tc_tiling_on_sc=False),
    )(table, src_idx, dst_idx, zer)


def _tc_h1_body(oa, x, wl, wr, bl, h1_out, h1h_out):
    o = oa[0].astype(jnp.float32) + oa[1].astype(jnp.float32)
    deg = o[:, 84:85]
    rd = 1.0 / jnp.maximum(deg, 1.0)
    a = o[:, :84] * rd
    h1 = (
        jnp.dot(a, wl[...], preferred_element_type=jnp.float32)
        + jnp.dot(x[...], wr[...], preferred_element_type=jnp.float32)
        + bl[...]
    )
    h1_out[...] = h1
    h1h_out[...] = h1.astype(jnp.bfloat16)


def _tc_h1(oa, x, wl, wr, bl):
    blk = NODES_PER_GRAPH
    grid = (N // blk,)
    return pl.pallas_call(
        _tc_h1_body,
        grid=grid,
        in_specs=[
            pl.BlockSpec((2, blk, 96), lambda i: (0, i, 0)),
            pl.BlockSpec((blk, 84), lambda i: (i, 0)),
            pl.BlockSpec((84, 128), lambda i: (0, 0)),
            pl.BlockSpec((84, 128), lambda i: (0, 0)),
            pl.BlockSpec((1, 128), lambda i: (0, 0)),
        ],
        out_specs=[
            pl.BlockSpec((blk, 128), lambda i: (i, 0)),
            pl.BlockSpec((blk, 128), lambda i: (i, 0)),
        ],
        out_shape=[
            jax.ShapeDtypeStruct((N, 128), jnp.float32),
            jax.ShapeDtypeStruct((N, 128), jnp.bfloat16),
        ],
    )(oa, x, wl, wr, bl)


def _tc_att_body(o2, oa, h1, pe, wl, wr, bl, aggr_out):
    agg2 = jnp.concatenate([o2[0], o2[1]], axis=-1).astype(jnp.float32)
    deg = (oa[0, :, 84:85].astype(jnp.float32)
           + oa[1, :, 84:85].astype(jnp.float32))
    rd = 1.0 / jnp.maximum(deg, 1.0)
    h1v = h1[...]
    h2 = (
        jnp.dot(agg2 * rd, wl[...], preferred_element_type=jnp.float32)
        + jnp.dot(h1v, wr[...], preferred_element_type=jnp.float32)
        + bl[...]
    )
    t2 = h2 + pe[...]
    t2b = t2.astype(jnp.bfloat16)
    t2s = (t2 * (1.0 / math.sqrt(128.0))).astype(jnp.bfloat16)
    score = lax.dot_general(t2s, t2b, (((1,), (1,)), ((), ())),
                            preferred_element_type=jnp.float32)
    p = jnp.exp(score)
    denom = jnp.sum(p, axis=-1, keepdims=True)
    ctx = jnp.dot(p.astype(jnp.bfloat16), t2b,
                  preferred_element_type=jnp.float32) / denom

    parts = []
    for tag in range(2):
        srcm = h1v if tag == 0 else ctx
        maxs = []
        means = []
        for j in range(12):
            blkv = srcm[84 * j:84 * (j + 1), :]
            maxs.append(jnp.max(blkv, axis=0, keepdims=True))
            means.append(jnp.mean(blkv, axis=0, keepdims=True))
        parts.append((jnp.concatenate(maxs, axis=0), jnp.concatenate(means, axis=0)))
    combined = jnp.concatenate([parts[0][0], parts[0][1], parts[1][0], parts[1][1]], axis=-1)
    aggr_out[...] = combined.reshape(1, 1, 6144)


def _tc_att(o2, o0, h1, pe, wl, wr, bl):
    blk = NODES_PER_GRAPH
    return pl.pallas_call(
        _tc_att_body,
        grid=(N_GRAPHS,),
        in_specs=[
            pl.BlockSpec((2, blk, 64), lambda g: (0, g, 0)),
            pl.BlockSpec((2, blk, 96), lambda g: (0, g, 0)),
            pl.BlockSpec((blk, 128), lambda g: (g, 0)),
            pl.BlockSpec((blk, 128), lambda g: (0, 0)),
            pl.BlockSpec((128, 128), lambda g: (0, 0)),
            pl.BlockSpec((128, 128), lambda g: (0, 0)),
            pl.BlockSpec((1, 128), lambda g: (0, 0)),
        ],
        out_specs=pl.BlockSpec((1, 1, 6144), lambda g: (g, 0, 0)),
        out_shape=jax.ShapeDtypeStruct((N_GRAPHS, 1, 6144), jnp.float32),
    )(o2, o0, h1, pe, wl, wr, bl)


def _tc_mlp_body(aggr, w1, b1, g1, be1, w2, b2, g2, be2, w3, b3, out):
    def bn(z, g, b):
        mm = jnp.mean(z, axis=0, keepdims=True)
        v = jnp.mean((z - mm) ** 2, axis=0, keepdims=True)
        return (z - mm) / jnp.sqrt(v + 1e-5) * g + b

    def silu(z):
        return z / (1.0 + jnp.exp(-z))

    z = silu(jnp.dot(aggr[...], w1[...], preferred_element_type=jnp.float32) + b1[...])
    z = bn(z, g1[...], be1[...])
    z = silu(jnp.dot(z, w2[...], preferred_element_type=jnp.float32) + b2[...])
    z = bn(z, g2[...], be2[...])
    logits = jnp.dot(z, w3[...], preferred_element_type=jnp.float32) + b3[...]
    l2 = logits[:, :2]
    lm = jnp.max(l2, axis=1, keepdims=True)
    e = jnp.exp(l2 - lm)
    sm = e / jnp.sum(e, axis=1, keepdims=True)
    out[...] = jnp.concatenate([sm, jnp.zeros((sm.shape[0], 126), jnp.float32)], axis=1)


def _tc_mlp(aggr, w1, b1, g1, be1, w2, b2, g2, be2, w3, b3):
    return pl.pallas_call(
        _tc_mlp_body,
        out_shape=jax.ShapeDtypeStruct((N_GRAPHS, 128), jnp.float32),
    )(aggr, w1, b1, g1, be1, w2, b2, g2, be2, w3, b3)


def kernel(x, edge_index, pe, Wl1, bl1, Wr1, Wl2, bl2, Wr2, W_fc1, b_fc1, g1,
           be1, W_fc2, b_fc2, g2, be2, W_fc3, b_fc3):
    f32 = jnp.float32
    src = edge_index[0]
    dst = edge_index[1]
    npad = EPAD - E
    src_pad = (jnp.arange(npad, dtype=jnp.int32) * 63) % N
    dst_pad = N + (jnp.arange(npad, dtype=jnp.int32) % NPAD)
    bf16 = jnp.bfloat16
    srcr = jnp.concatenate([src, src_pad]).reshape(NS, NCH, CH)
    dstr = jnp.concatenate([dst, dst_pad]).reshape(NS, NCH, CH)

    ones = jnp.ones((N, 1), f32)
    table1 = jnp.concatenate([x, ones, jnp.zeros((N, 11), f32)], axis=1).astype(bf16)
    zer96 = jnp.zeros((ZROWS, 96), bf16)
    zer64 = jnp.zeros((ZROWS, 64), bf16)

    oA = _sc_segsum_l1(table1, srcr, dstr, zer96)

    h1, h1b = _tc_h1(oA, x, Wl1, Wr1, bl1.reshape(1, 128))

    o2 = _sc_segsum_l2(h1b.reshape(2 * N, 64), srcr, dstr, zer64)

    aggr = _tc_att(o2, oA, h1, pe, Wl2, Wr2, bl2.reshape(1, 128)).reshape(N_GRAPHS, 6144)

    w2p = jnp.pad(W_fc2, ((0, 0), (0, 96)))
    b2p = jnp.pad(b_fc2, (0, 96)).reshape(1, 128)
    g2p = jnp.pad(g2, (0, 96)).reshape(1, 128)
    be2p = jnp.pad(be2, (0, 96)).reshape(1, 128)
    w3p = jnp.pad(W_fc3, ((0, 96), (0, 126)))
    b3p = jnp.pad(b_fc3, (0, 126)).reshape(1, 128)
    outp = _tc_mlp(aggr, W_fc1, b_fc1.reshape(1, 512), g1.reshape(1, 512),
                   be1.reshape(1, 512), w2p, b2p, g2p, be2p, w3p, b3p)
    return outp[:, :2]
